# baseline (speedup 1.0000x reference)
import functools

import jax
import jax.numpy as jnp
from jax import lax
from jax.experimental import pallas as pl
from jax.experimental.pallas import tpu as pltpu
from jax.experimental.pallas import tpu_sc as plsc

B = 4
S = 8192
S1 = S + 1
D = 1024
HALF = D // 2
NPOS = 101
NC, NS = 2, 16
NW = NC * NS
JPT = S // NW
JCH = 8
NCHUNK = JPT // JCH
BASER_LEN = 1152
PSTG = 264
MCH = 2048


def _stage_off(t, b):
    return (t * B + b) * PSTG


def _body(x_hbm, p0_hbm, p1_hbm, p2_hbm, pe_hbm, out_hbm,
          petab, baser, xbuf, pstage, sin0, sin1, sout0, sout1):
    wid = lax.axis_index("s") * NC + lax.axis_index("c")
    iota = lax.iota(jnp.int32, 16)
    phbm = (p0_hbm, p1_hbm, p2_hbm)

    pltpu.sync_copy(pe_hbm, petab)

    for b in range(B):
        def scan_chunk(c, mv):
            off = pl.multiple_of(b * S + c * MCH, MCH)
            pltpu.sync_copy(p0_hbm.at[pl.ds(off, MCH)],
                            pstage.at[pl.ds(0, MCH)])

            def vmax(u, mv2):
                return jnp.maximum(
                    mv2, pstage[pl.ds(pl.multiple_of(u * 16, 16), 16)])

            return lax.fori_loop(0, MCH // 16, vmax, mv)

        mvec = lax.fori_loop(0, S // MCH, scan_chunk,
                             jnp.full((16,), -1, jnp.int32))
        m = jnp.max(mvec)
        step = m + 1

        def fill(u, _):
            t = iota + u * 16
            d = (D - 1) - t
            cond = (lax.rem(d, step) == 0) & (d >= 0) & (m > 0)
            baser[pl.ds(pl.multiple_of(b * BASER_LEN + u * 16, 16),
                        16)] = jnp.where(cond, 1.0, 0.0).astype(jnp.float32)
            return 0

        lax.fori_loop(0, BASER_LEN // 16, fill, 0)

    pbase = pl.multiple_of(jnp.maximum(wid * JPT - 8, 0), 8)
    boff = wid * JPT - 1 - pbase
    for t in range(3):
        for b in range(B):
            pltpu.sync_copy(
                phbm[t].at[pl.ds(pl.multiple_of(b * S + pbase, 8), PSTG)],
                pstage.at[pl.ds(_stage_off(t, b), PSTG)])

    j_lo = wid * JPT

    def compute_chunk(k, buf):
        start_i = jnp.where((wid == 0) & (k == 0), 1, 0)

        def row(i, _):
            lr = k * JCH + i
            for b in range(B):
                pidx = jnp.full((16,), 0, jnp.int32) + (boff + lr)
                p0 = plsc.load_gather(pstage, [pidx + _stage_off(0, b)])
                p1 = plsc.load_gather(pstage, [pidx + _stage_off(1, b)])
                p2 = plsc.load_gather(pstage, [pidx + _stage_off(2, b)])
                sbase = p0 + b * BASER_LEN
                for v in range(D // 16):
                    ccol = iota + v * 16
                    if v < HALF // 16:
                        pev = plsc.load_gather(petab, [p1 * HALF + ccol])
                    else:
                        pev = plsc.load_gather(petab,
                                               [p2 * HALF + (ccol - HALF)])
                    secv = plsc.load_gather(baser, [sbase + ccol])
                    q = (v // 8) * B + b
                    l0 = (v % 8) * 16
                    xv = xbuf[buf, i, q, pl.ds(l0, 16)]
                    xbuf[buf, i, q, pl.ds(l0, 16)] = xv + pev + secv
            return 0

        lax.fori_loop(start_i, JCH, row, 0)

    def start_in(k, buf, sem):
        pltpu.async_copy(x_hbm.at[pl.ds(j_lo + k * JCH, JCH), :, :],
                         xbuf.at[buf], sem)

    def wait_in(buf, sem):
        pltpu.make_async_copy(x_hbm.at[pl.ds(0, JCH), :, :],
                              xbuf.at[buf], sem).wait()

    def start_out(k, buf, sem):
        pltpu.async_copy(xbuf.at[buf],
                         out_hbm.at[pl.ds(j_lo + k * JCH, JCH), :, :], sem)

    def wait_out(buf, sem):
        pltpu.make_async_copy(xbuf.at[buf],
                              out_hbm.at[pl.ds(0, JCH), :, :], sem).wait()

    start_in(0, 0, sin0)
    start_in(1, 1, sin1)

    def pair(j2, _):
        k0 = j2 * 2
        wait_in(0, sin0)
        compute_chunk(k0, 0)
        start_out(k0, 0, sout0)
        wait_in(1, sin1)
        compute_chunk(k0 + 1, 1)
        start_out(k0 + 1, 1, sout1)

        @pl.when(j2 < NCHUNK // 2 - 1)
        def _prefetch():
            wait_out(0, sout0)
            start_in(k0 + 2, 0, sin0)
            wait_out(1, sout1)
            start_in(k0 + 3, 1, sin1)

        return 0

    lax.fori_loop(0, NCHUNK // 2, pair, 0)
    wait_out(0, sout0)
    wait_out(1, sout1)

    @pl.when(wid == NW - 1)
    def _last_row():
        for b in range(B):
            pltpu.sync_copy(p0_hbm.at[pl.ds(b * S + S - 8, 8)],
                            pstage.at[pl.ds(b * 8, 8)])
            pltpu.sync_copy(p1_hbm.at[pl.ds(b * S + S - 8, 8)],
                            pstage.at[pl.ds(32 + b * 8, 8)])
            pltpu.sync_copy(p2_hbm.at[pl.ds(b * S + S - 8, 8)],
                            pstage.at[pl.ds(64 + b * 8, 8)])
        pltpu.sync_copy(x_hbm.at[pl.ds(S, 1), :, :],
                        xbuf.at[0, pl.ds(0, 1)])
        pidx7 = jnp.full((16,), 7, jnp.int32)
        for b in range(B):
            p0 = plsc.load_gather(pstage, [pidx7 + b * 8])
            p1 = plsc.load_gather(pstage, [pidx7 + 32 + b * 8])
            p2 = plsc.load_gather(pstage, [pidx7 + 64 + b * 8])
            sbase = p0 + b * BASER_LEN
            for v in range(D // 16):
                ccol = iota + v * 16
                if v < HALF // 16:
                    pev = plsc.load_gather(petab, [p1 * HALF + ccol])
                else:
                    pev = plsc.load_gather(petab, [p2 * HALF + (ccol - HALF)])
                secv = plsc.load_gather(baser, [sbase + ccol])
                q = (v // 8) * B + b
                l0 = (v % 8) * 16
                xv = xbuf[0, 0, q, pl.ds(l0, 16)]
                xbuf[0, 0, q, pl.ds(l0, 16)] = xv + pev + secv
        pltpu.sync_copy(xbuf.at[0, pl.ds(0, 1)],
                        out_hbm.at[pl.ds(S, 1), :, :])


_sc_call = functools.partial(
    pl.kernel,
    out_type=jax.ShapeDtypeStruct((S1, (D // 128) * B, 128), jnp.float32),
    mesh=plsc.VectorSubcoreMesh(core_axis_name="c", subcore_axis_name="s"),
    compiler_params=pltpu.CompilerParams(use_tc_tiling_on_sc=True,
                                         needs_layout_passes=False),
    scratch_types=[
        pltpu.VMEM((NPOS * HALF,), jnp.float32),
        pltpu.VMEM((B * BASER_LEN,), jnp.float32),
        pltpu.VMEM((2, JCH, (D // 128) * B, 128), jnp.float32),
        pltpu.VMEM((3 * B * PSTG,), jnp.int32),
        pltpu.SemaphoreType.DMA,
        pltpu.SemaphoreType.DMA,
        pltpu.SemaphoreType.DMA,
        pltpu.SemaphoreType.DMA,
    ],
)(_body)


def kernel(x, pos, pos_embed):
    posr = pos.astype(jnp.int32)
    p0 = posr[:, :, 0].reshape(B * S)
    p1 = posr[:, :, 1].reshape(B * S)
    p2 = posr[:, :, 2].reshape(B * S)
    pe1 = pos_embed.reshape(NPOS * HALF)
    xv = (x.transpose(1, 2, 0)
           .reshape(S1, D // 128, 128, B)
           .transpose(0, 1, 3, 2)
           .reshape(S1, (D // 128) * B, 128))
    ov = _sc_call(xv, p0, p1, p2, pe1)
    return (ov.reshape(S1, D // 128, B, 128)
              .transpose(0, 1, 3, 2)
              .reshape(S1, D, B)
              .transpose(2, 0, 1))


# speedup vs baseline: 2.2265x; 2.2265x over previous
import functools

import jax
import jax.numpy as jnp
from jax import lax
from jax.experimental import pallas as pl
from jax.experimental.pallas import tpu as pltpu
from jax.experimental.pallas import tpu_sc as plsc

B = 4
S = 8192
S1 = S + 1
D = 1024
HALF = D // 2
NPOS = 101
NC, NS = 2, 16
NW = NC * NS
JPT = S // NW
JCH = 4
NCHUNK = JPT // JCH
NBUF = 4
Q = (D // 128) * B
BASER_LEN = 1152
MCH = 2048


def _body(x_hbm, p0_hbm, p1_hbm, p2_hbm, pe_hbm, out_hbm,
          petab, baser, xbuf, pstage, *sems):
    sin = sems[:NBUF]
    sout = sems[NBUF:]
    wid = lax.axis_index("s") * NC + lax.axis_index("c")
    iota = lax.iota(jnp.int32, 16)
    phbm = (p0_hbm, p1_hbm, p2_hbm)
    j_lo = wid * JPT + 1

    def start_in(k, bb):
        pltpu.async_copy(x_hbm.at[pl.ds(j_lo + k * JCH, JCH), :, :],
                         xbuf.at[bb], sin[bb])

    def wait_in(bb):
        pltpu.make_async_copy(x_hbm.at[pl.ds(0, JCH), :, :],
                              xbuf.at[bb], sin[bb]).wait()

    def start_out(k, bb):
        pltpu.async_copy(xbuf.at[bb],
                         out_hbm.at[pl.ds(j_lo + k * JCH, JCH), :, :],
                         sout[bb])

    def wait_out(bb):
        pltpu.make_async_copy(xbuf.at[bb],
                              out_hbm.at[pl.ds(0, JCH), :, :],
                              sout[bb]).wait()

    start_in(0, 0)
    start_in(1, 1)

    pltpu.sync_copy(pe_hbm, petab)

    for b in range(B):
        def scan_chunk(c, mv):
            off = pl.multiple_of(b * S + c * MCH, MCH)
            pltpu.sync_copy(p0_hbm.at[pl.ds(off, MCH)],
                            pstage.at[pl.ds(0, MCH)])

            def vmax(u, mv2):
                return jnp.maximum(
                    mv2, pstage[pl.ds(pl.multiple_of(u * 16, 16), 16)])

            return lax.fori_loop(0, MCH // 16, vmax, mv)

        mvec = lax.fori_loop(0, S // MCH, scan_chunk,
                             jnp.full((16,), -1, jnp.int32))
        m = jnp.max(mvec)
        step = m + 1

        def fill(u, _):
            t = iota + u * 16
            d = (D - 1) - t
            cond = (lax.rem(d, step) == 0) & (d >= 0) & (m > 0)
            baser[pl.ds(pl.multiple_of(b * BASER_LEN + u * 16, 16),
                        16)] = jnp.where(cond, 1.0, 0.0).astype(jnp.float32)
            return 0

        lax.fori_loop(0, BASER_LEN // 16, fill, 0)

    for t in range(3):
        for b in range(B):
            src = pl.multiple_of(b * S + wid * JPT, 8)
            pltpu.sync_copy(phbm[t].at[pl.ds(src, JPT)],
                            pstage.at[pl.ds((t * B + b) * JPT, JPT)])

    def compute_chunk(k, bb):
        def row(i2, _):
            i = i2 >> 2
            b = i2 & (B - 1)
            lr = k * JCH + i
            pidx = jnp.full((16,), 0, jnp.int32) + lr + b * JPT
            p0 = plsc.load_gather(pstage, [pidx])
            p1 = plsc.load_gather(pstage, [pidx + B * JPT])
            p2 = plsc.load_gather(pstage, [pidx + 2 * B * JPT])
            sbase = p0 + b * BASER_LEN
            for v in range(D // 16):
                ccol = iota + v * 16
                if v < HALF // 16:
                    pev = plsc.load_gather(petab, [p1 * HALF + ccol])
                else:
                    pev = plsc.load_gather(petab, [p2 * HALF + (ccol - HALF)])
                secv = plsc.load_gather(baser, [sbase + ccol])
                q = (v // 8) * B + b
                l0 = (v % 8) * 16
                xv = xbuf[bb, i, q, pl.ds(l0, 16)]
                xbuf[bb, i, q, pl.ds(l0, 16)] = xv + pev + secv
            return 0

        lax.fori_loop(0, JCH * B, row, 0)

    def group(g, _):
        for bb in range(NBUF):
            k = g * NBUF + bb
            wait_in(bb)
            compute_chunk(k, bb)
            start_out(k, bb)
            nb = (bb + 2) % NBUF

            @pl.when(k >= 2)
            def _drain():
                wait_out(nb)

            @pl.when(k + 2 < NCHUNK)
            def _refill():
                start_in(k + 2, nb)

        return 0

    lax.fori_loop(0, NCHUNK // NBUF, group, 0)
    wait_out((NCHUNK - 2) % NBUF)
    wait_out((NCHUNK - 1) % NBUF)

    @pl.when(wid == NW - 1)
    def _copy_row0():
        pltpu.sync_copy(x_hbm.at[pl.ds(0, 1), :, :], xbuf.at[0, pl.ds(0, 1)])
        pltpu.sync_copy(xbuf.at[0, pl.ds(0, 1)], out_hbm.at[pl.ds(0, 1), :, :])


_sc_call = functools.partial(
    pl.kernel,
    out_type=jax.ShapeDtypeStruct((S1, Q, 128), jnp.float32),
    mesh=plsc.VectorSubcoreMesh(core_axis_name="c", subcore_axis_name="s"),
    compiler_params=pltpu.CompilerParams(use_tc_tiling_on_sc=True,
                                         needs_layout_passes=False),
    scratch_types=[
        pltpu.VMEM((NPOS * HALF,), jnp.float32),
        pltpu.VMEM((B * BASER_LEN,), jnp.float32),
        pltpu.VMEM((NBUF, JCH, Q, 128), jnp.float32),
        pltpu.VMEM((3 * B * JPT,), jnp.int32),
    ] + [pltpu.SemaphoreType.DMA] * (2 * NBUF),
)(_body)


def kernel(x, pos, pos_embed):
    posr = pos.astype(jnp.int32)
    p0 = posr[:, :, 0].reshape(B * S)
    p1 = posr[:, :, 1].reshape(B * S)
    p2 = posr[:, :, 2].reshape(B * S)
    pe1 = pos_embed.reshape(NPOS * HALF)
    xv = (x.transpose(1, 2, 0)
           .reshape(S1, D // 128, 128, B)
           .transpose(0, 1, 3, 2)
           .reshape(S1, Q, 128))
    ov = _sc_call(xv, p0, p1, p2, pe1)
    return (ov.reshape(S1, D // 128, B, 128)
              .transpose(0, 1, 3, 2)
              .reshape(S1, D, B)
              .transpose(2, 0, 1))
